# Initial kernel scaffold; baseline (speedup 1.0000x reference)
import jax
import jax.numpy as jnp
from jax.experimental import pallas as pl


def kernel(q_points, s_points, neighb_inds, x, W1, bn1_g, bn1_b, Wc, bnc_g, bnc_b, bno_g, bno_b, W2, bn2_g, bn2_b):
    raise NotImplementedError("write your pallas kernel here")


# speedup vs baseline: 4.2783x; 4.2783x over previous
import functools

import jax
import jax.numpy as jnp
from jax import lax
from jax.experimental import pallas as pl
from jax.experimental.pallas import tpu as pltpu
from jax.experimental.pallas import tpu_sc as plsc

N = 10000
K = 32
IN_DIM = 128
OUT_DIM = 128
HID = 32
EPS = 1e-5

NW = 32
NPAD = 10240
PW = NPAD // NW
C = 32
NCH = PW // C
FIRE = 128
FPC = C * K // FIRE
NT = N + 16


def _lrelu(v):
    return jnp.where(v >= 0, v, 0.1 * v)


def _tc1_body(x_ref, w1_ref, g1_ref, b1_ref, wc_ref, out_ref):
    x = x_ref[...]
    h = lax.dot_general(x, w1_ref[...], (((1,), (1,)), ((), ())),
                        preferred_element_type=jnp.float32)
    m = jnp.mean(h, axis=0, keepdims=True)
    v = jnp.mean((h - m) * (h - m), axis=0, keepdims=True)
    h = _lrelu((h - m) / jnp.sqrt(v + EPS) * g1_ref[...] + b1_ref[...])
    h2 = lax.dot_general(h, wc_ref[...], (((1,), (1,)), ((), ())),
                         preferred_element_type=jnp.float32)
    out_ref[:N, :] = h2
    out_ref[N:, :] = jnp.zeros((NT - N, HID), jnp.float32)


def _tc2_body(mx_ref, mn_ref, st_ref, x_ref, cg_ref, cb_ref, og_ref, ob_ref,
              w2_ref, g2_ref, b2_ref, out_ref):
    s = jnp.sum(st_ref[...], axis=0, keepdims=True)
    cnt = float(N * K)
    meanc = s[:, :HID] / cnt
    varc = s[:, HID:] / cnt - meanc * meanc
    scale = cg_ref[...] / jnp.sqrt(varc + EPS)
    shift = cb_ref[...] - meanc * scale
    picked = jnp.where(scale >= 0, mx_ref[...], mn_ref[...])
    v = _lrelu(picked * scale + shift)
    m = jnp.mean(v, axis=0, keepdims=True)
    var = jnp.mean((v - m) * (v - m), axis=0, keepdims=True)
    v = _lrelu((v - m) / jnp.sqrt(var + EPS) * og_ref[...] + ob_ref[...])
    y = lax.dot_general(v, w2_ref[...], (((1,), (1,)), ((), ())),
                        preferred_element_type=jnp.float32)
    m2 = jnp.mean(y, axis=0, keepdims=True)
    var2 = jnp.mean((y - m2) * (y - m2), axis=0, keepdims=True)
    y = (y - m2) / jnp.sqrt(var2 + EPS) * g2_ref[...] + b2_ref[...]
    out_ref[...] = _lrelu(y + x_ref[...])


def _sc_body(h2_hbm, inds_hbm, mx_hbm, mn_hbm, st_hbm,
             idx_v, rows_v, obx_v, obn_v, acc_v, idx_sem, g_sem):
    cid = lax.axis_index("c")
    sid = lax.axis_index("s")
    w = sid * 2 + cid
    node0 = w * PW
    irow0 = w * (PW * K // FIRE)

    def fire(b):
        hs = []
        for j in range(FPC):
            hs.append(pltpu.async_copy(
                h2_hbm.at[idx_v.at[b, j]],
                rows_v.at[b, pl.ds(j * FIRE, FIRE)],
                g_sem))
        return hs

    pltpu.sync_copy(inds_hbm.at[pl.ds(irow0, FPC)], idx_v.at[0])
    gh = fire(0)
    ih = pltpu.async_copy(inds_hbm.at[pl.ds(irow0 + FPC, FPC)],
                          idx_v.at[1], idx_sem)

    zeros = jnp.zeros((16,), jnp.float32)
    s0, s1, q0, q1 = zeros, zeros, zeros, zeros
    for g in range(NCH):
        b = g % 2
        for h in gh:
            h.wait()
        if g + 1 < NCH:
            ih.wait()
            gh = fire((g + 1) % 2)
            if g + 2 < NCH:
                ih = pltpu.async_copy(
                    inds_hbm.at[pl.ds(irow0 + (g + 2) * FPC, FPC)],
                    idx_v.at[b], idx_sem)

        def node(i, carry):
            s0, s1, q0, q1 = carry
            base = i * K
            r0 = rows_v[b, base, pl.ds(0, 16)]
            r1 = rows_v[b, base, pl.ds(16, 16)]
            mx0, mn0, mx1, mn1 = r0, r0, r1, r1
            s0 = s0 + r0
            s1 = s1 + r1
            q0 = q0 + r0 * r0
            q1 = q1 + r1 * r1
            for j in range(1, K):
                r0 = rows_v[b, base + j, pl.ds(0, 16)]
                r1 = rows_v[b, base + j, pl.ds(16, 16)]
                mx0 = jnp.maximum(mx0, r0)
                mn0 = jnp.minimum(mn0, r0)
                mx1 = jnp.maximum(mx1, r1)
                mn1 = jnp.minimum(mn1, r1)
                s0 = s0 + r0
                s1 = s1 + r1
                q0 = q0 + r0 * r0
                q1 = q1 + r1 * r1
            obx_v[b, i, pl.ds(0, 16)] = mx0
            obx_v[b, i, pl.ds(16, 16)] = mx1
            obn_v[b, i, pl.ds(0, 16)] = mn0
            obn_v[b, i, pl.ds(16, 16)] = mn1
            return (s0, s1, q0, q1)

        s0, s1, q0, q1 = lax.fori_loop(0, C, node, (s0, s1, q0, q1))
        pltpu.sync_copy(obx_v.at[b], mx_hbm.at[pl.ds(node0 + g * C, C)])
        pltpu.sync_copy(obn_v.at[b], mn_hbm.at[pl.ds(node0 + g * C, C)])

    acc_v[0, pl.ds(0, 16)] = s0
    acc_v[0, pl.ds(16, 16)] = s1
    acc_v[0, pl.ds(32, 16)] = q0
    acc_v[0, pl.ds(48, 16)] = q1
    zz = jnp.zeros((16,), jnp.float32)
    for r in range(1, 8):
        for c in range(4):
            acc_v[r, pl.ds(c * 16, 16)] = zz
    pltpu.sync_copy(acc_v, st_hbm.at[pl.ds(w * 8, 8)])


_sc_call = functools.partial(
    pl.kernel,
    mesh=plsc.VectorSubcoreMesh(core_axis_name="c", subcore_axis_name="s"),
    out_type=[
        jax.ShapeDtypeStruct((NPAD, HID), jnp.float32),
        jax.ShapeDtypeStruct((NPAD, HID), jnp.float32),
        jax.ShapeDtypeStruct((NW * 8, 2 * HID), jnp.float32),
    ],
    scratch_types=[
        pltpu.VMEM((2, FPC, FIRE), jnp.int32),
        pltpu.VMEM((2, C * K, HID), jnp.float32),
        pltpu.VMEM((2, C, HID), jnp.float32),
        pltpu.VMEM((2, C, HID), jnp.float32),
        pltpu.VMEM((8, 2 * HID), jnp.float32),
        pltpu.SemaphoreType.DMA,
        pltpu.SemaphoreType.DMA,
    ],
    compiler_params=pltpu.CompilerParams(use_tc_tiling_on_sc=False),
)(_sc_body)


def kernel(q_points, s_points, neighb_inds, x,
           W1, bn1_g, bn1_b, Wc, bnc_g, bnc_b, bno_g, bno_b, W2, bn2_g, bn2_b):
    h2 = pl.pallas_call(
        _tc1_body,
        out_shape=jax.ShapeDtypeStruct((NT, HID), jnp.float32),
    )(x, W1, bn1_g.reshape(1, HID), bn1_b.reshape(1, HID), Wc)

    inds = jnp.pad(neighb_inds, ((0, NPAD - N), (0, 0)), constant_values=N)
    inds = inds.reshape(NPAD * K // FIRE, FIRE)
    mx, mn, st = _sc_call(h2, inds)

    out = pl.pallas_call(
        _tc2_body,
        out_shape=jax.ShapeDtypeStruct((N, OUT_DIM), jnp.float32),
    )(mx[:N], mn[:N], st, x,
      bnc_g.reshape(1, HID), bnc_b.reshape(1, HID),
      bno_g.reshape(1, HID), bno_b.reshape(1, HID),
      W2, bn2_g.reshape(1, OUT_DIM), bn2_b.reshape(1, OUT_DIM))
    return out


# speedup vs baseline: 5.4723x; 1.2791x over previous
import functools

import jax
import jax.numpy as jnp
import numpy as np
from jax import lax
from jax.experimental import pallas as pl
from jax.experimental.pallas import tpu as pltpu
from jax.experimental.pallas import tpu_sc as plsc

N = 10000
K = 32
IN_DIM = 128
OUT_DIM = 128
HID = 32
EPS = 1e-5

NW = 32
NPAD = 10240
PW = NPAD // NW
C = 32
NCH = PW // C
FIRE = 128
FPC = C * K // FIRE
NT = N + 16

PERM = np.concatenate([np.arange(0, HID, 2), np.arange(1, HID, 2)])


def _lrelu(v):
    return jnp.where(v >= 0, v, 0.1 * v)


def _tc1_body(x_ref, w1_ref, g1_ref, b1_ref, wc_ref, out_ref):
    x = x_ref[...]
    h = lax.dot_general(x, w1_ref[...], (((1,), (1,)), ((), ())),
                        preferred_element_type=jnp.float32)
    m = jnp.mean(h, axis=0, keepdims=True)
    v = jnp.mean((h - m) * (h - m), axis=0, keepdims=True)
    h = _lrelu((h - m) / jnp.sqrt(v + EPS) * g1_ref[...] + b1_ref[...])
    h2 = lax.dot_general(h, wc_ref[...], (((1,), (1,)), ((), ())),
                         preferred_element_type=jnp.float32)
    out_ref[:N, :] = h2.astype(jnp.bfloat16)
    out_ref[N:, :] = jnp.zeros((NT - N, HID), jnp.bfloat16)


def _tc2_body(mx_ref, mn_ref, st_ref, x_ref, cg_ref, cb_ref, og_ref, ob_ref,
              w2_ref, g2_ref, b2_ref, out_ref):
    s = jnp.sum(st_ref[...], axis=0, keepdims=True)
    cnt = float(N * K)
    meanc = s[:, :HID] / cnt
    varc = s[:, HID:] / cnt - meanc * meanc
    scale = cg_ref[...] / jnp.sqrt(varc + EPS)
    shift = cb_ref[...] - meanc * scale
    picked = jnp.where(scale >= 0, mx_ref[...], mn_ref[...])
    v = _lrelu(picked * scale + shift)
    m = jnp.mean(v, axis=0, keepdims=True)
    var = jnp.mean((v - m) * (v - m), axis=0, keepdims=True)
    v = _lrelu((v - m) / jnp.sqrt(var + EPS) * og_ref[...] + ob_ref[...])
    y = lax.dot_general(v, w2_ref[...], (((1,), (1,)), ((), ())),
                        preferred_element_type=jnp.float32)
    m2 = jnp.mean(y, axis=0, keepdims=True)
    var2 = jnp.mean((y - m2) * (y - m2), axis=0, keepdims=True)
    y = (y - m2) / jnp.sqrt(var2 + EPS) * g2_ref[...] + b2_ref[...]
    out_ref[...] = _lrelu(y + x_ref[...])


def _sc_body(h2_hbm, inds_hbm, mx_hbm, mn_hbm, st_hbm,
             idx_v, rows_v, obx_v, obn_v, acc_v, idx_sem, g_sem):
    cid = lax.axis_index("c")
    sid = lax.axis_index("s")
    w = sid * 2 + cid
    node0 = w * PW
    irow0 = w * (PW * K // FIRE)

    def fire(b):
        hs = []
        for j in range(FPC):
            hs.append(pltpu.async_copy(
                h2_hbm.at[idx_v.at[b, j]],
                rows_v.at[b, pl.ds(j * FIRE, FIRE)],
                g_sem))
        return hs

    pltpu.sync_copy(inds_hbm.at[pl.ds(irow0, FPC)], idx_v.at[0])
    gh = fire(0)
    ih = pltpu.async_copy(inds_hbm.at[pl.ds(irow0 + FPC, FPC)],
                          idx_v.at[1], idx_sem)

    zeros = jnp.zeros((16,), jnp.float32)
    s0, s1, q0, q1 = zeros, zeros, zeros, zeros
    for g in range(NCH):
        b = g % 2
        for h in gh:
            h.wait()
        if g + 1 < NCH:
            ih.wait()
            gh = fire((g + 1) % 2)
            if g + 2 < NCH:
                ih = pltpu.async_copy(
                    inds_hbm.at[pl.ds(irow0 + (g + 2) * FPC, FPC)],
                    idx_v.at[b], idx_sem)

        def node(i, carry):
            s0, s1, q0, q1 = carry
            base = i * K
            r0, r1 = plsc.unpack(rows_v[b, base],
                                 format=plsc.PackFormat.INTERLEAVED)
            mx0, mn0, mx1, mn1 = r0, r0, r1, r1
            s0 = s0 + r0
            s1 = s1 + r1
            q0 = q0 + r0 * r0
            q1 = q1 + r1 * r1
            for j in range(1, K):
                r0, r1 = plsc.unpack(rows_v[b, base + j],
                                     format=plsc.PackFormat.INTERLEAVED)
                mx0 = jnp.maximum(mx0, r0)
                mn0 = jnp.minimum(mn0, r0)
                mx1 = jnp.maximum(mx1, r1)
                mn1 = jnp.minimum(mn1, r1)
                s0 = s0 + r0
                s1 = s1 + r1
                q0 = q0 + r0 * r0
                q1 = q1 + r1 * r1
            obx_v[b, i, pl.ds(0, 16)] = mx0
            obx_v[b, i, pl.ds(16, 16)] = mx1
            obn_v[b, i, pl.ds(0, 16)] = mn0
            obn_v[b, i, pl.ds(16, 16)] = mn1
            return (s0, s1, q0, q1)

        s0, s1, q0, q1 = lax.fori_loop(0, C, node, (s0, s1, q0, q1))
        pltpu.sync_copy(obx_v.at[b], mx_hbm.at[pl.ds(node0 + g * C, C)])
        pltpu.sync_copy(obn_v.at[b], mn_hbm.at[pl.ds(node0 + g * C, C)])

    acc_v[0, pl.ds(0, 16)] = s0
    acc_v[0, pl.ds(16, 16)] = s1
    acc_v[0, pl.ds(32, 16)] = q0
    acc_v[0, pl.ds(48, 16)] = q1
    zz = jnp.zeros((16,), jnp.float32)
    for r in range(1, 8):
        for c in range(4):
            acc_v[r, pl.ds(c * 16, 16)] = zz
    pltpu.sync_copy(acc_v, st_hbm.at[pl.ds(w * 8, 8)])


_sc_call = functools.partial(
    pl.kernel,
    mesh=plsc.VectorSubcoreMesh(core_axis_name="c", subcore_axis_name="s"),
    out_type=[
        jax.ShapeDtypeStruct((NPAD, HID), jnp.float32),
        jax.ShapeDtypeStruct((NPAD, HID), jnp.float32),
        jax.ShapeDtypeStruct((NW * 8, 2 * HID), jnp.float32),
    ],
    scratch_types=[
        pltpu.VMEM((2, FPC, FIRE), jnp.int32),
        pltpu.VMEM((2, C * K, HID), jnp.bfloat16),
        pltpu.VMEM((2, C, HID), jnp.float32),
        pltpu.VMEM((2, C, HID), jnp.float32),
        pltpu.VMEM((8, 2 * HID), jnp.float32),
        pltpu.SemaphoreType.DMA,
        pltpu.SemaphoreType.DMA,
    ],
    compiler_params=pltpu.CompilerParams(use_tc_tiling_on_sc=False,
                                         needs_layout_passes=False),
)(_sc_body)


def kernel(q_points, s_points, neighb_inds, x,
           W1, bn1_g, bn1_b, Wc, bnc_g, bnc_b, bno_g, bno_b, W2, bn2_g, bn2_b):
    h2 = pl.pallas_call(
        _tc1_body,
        out_shape=jax.ShapeDtypeStruct((NT, HID), jnp.bfloat16),
    )(x, W1, bn1_g.reshape(1, HID), bn1_b.reshape(1, HID), Wc)

    inds = jnp.pad(neighb_inds, ((0, NPAD - N), (0, 0)), constant_values=N)
    inds = inds.reshape(NPAD * K // FIRE, FIRE)
    mx, mn, st = _sc_call(h2, inds)

    out = pl.pallas_call(
        _tc2_body,
        out_shape=jax.ShapeDtypeStruct((N, OUT_DIM), jnp.float32),
    )(mx[:N], mn[:N], st, x,
      bnc_g[PERM].reshape(1, HID), bnc_b[PERM].reshape(1, HID),
      bno_g[PERM].reshape(1, HID), bno_b[PERM].reshape(1, HID),
      W2[:, PERM], bn2_g.reshape(1, OUT_DIM), bn2_b.reshape(1, OUT_DIM))
    return out


# speedup vs baseline: 7.8989x; 1.4434x over previous
import functools

import jax
import jax.numpy as jnp
import numpy as np
from jax import lax
from jax.experimental import pallas as pl
from jax.experimental.pallas import tpu as pltpu
from jax.experimental.pallas import tpu_sc as plsc

N = 10000
K = 32
IN_DIM = 128
OUT_DIM = 128
HID = 32
EPS = 1e-5

NW = 32
NPAD = 10240
PW = NPAD // NW
C = 32
NCH = PW // C
FIRE = 128
FPC = C * K // FIRE
NT = N + 16

PERM = np.concatenate([np.arange(0, HID, 2), np.arange(1, HID, 2)])


def _lrelu(v):
    return jnp.where(v >= 0, v, 0.1 * v)


def _tc1_body(x_ref, w1_ref, g1_ref, b1_ref, wc_ref, out_ref):
    x = x_ref[...]
    h = lax.dot_general(x, w1_ref[...], (((1,), (1,)), ((), ())),
                        preferred_element_type=jnp.float32)
    m = jnp.mean(h, axis=0, keepdims=True)
    v = jnp.mean((h - m) * (h - m), axis=0, keepdims=True)
    h = _lrelu((h - m) / jnp.sqrt(v + EPS) * g1_ref[...] + b1_ref[...])
    h2 = lax.dot_general(h, wc_ref[...], (((1,), (1,)), ((), ())),
                         preferred_element_type=jnp.float32)
    out_ref[:N, :] = h2.astype(jnp.bfloat16)
    out_ref[N:, :] = jnp.zeros((NT - N, HID), jnp.bfloat16)


def _tc2_body(mx_ref, mn_ref, st_ref, x_ref, cg_ref, cb_ref, og_ref, ob_ref,
              w2_ref, g2_ref, b2_ref, out_ref):
    s = jnp.sum(st_ref[...], axis=0, keepdims=True)
    cnt = float(N * K)
    meanc = s[:, :HID] / cnt
    varc = s[:, HID:] / cnt - meanc * meanc
    scale = cg_ref[...] / jnp.sqrt(varc + EPS)
    shift = cb_ref[...] - meanc * scale
    picked = jnp.where(scale >= 0, mx_ref[...], mn_ref[...])
    v = _lrelu(picked * scale + shift)
    m = jnp.mean(v, axis=0, keepdims=True)
    var = jnp.mean((v - m) * (v - m), axis=0, keepdims=True)
    v = _lrelu((v - m) / jnp.sqrt(var + EPS) * og_ref[...] + ob_ref[...])
    y = lax.dot_general(v, w2_ref[...], (((1,), (1,)), ((), ())),
                        preferred_element_type=jnp.float32)
    m2 = jnp.mean(y, axis=0, keepdims=True)
    var2 = jnp.mean((y - m2) * (y - m2), axis=0, keepdims=True)
    y = (y - m2) / jnp.sqrt(var2 + EPS) * g2_ref[...] + b2_ref[...]
    out_ref[...] = _lrelu(y + x_ref[...])


def _sc_body(h2_hbm, inds_hbm, mx_hbm, mn_hbm, st_hbm,
             idx_v, rows_v, obx_v, obn_v, acc_v, tab_sh, idx_sem, g_sem):
    cid = lax.axis_index("c")
    sid = lax.axis_index("s")
    w = sid * 2 + cid
    node0 = w * PW
    irow0 = w * (PW * K // FIRE)

    @pl.when(sid == 0)
    def _stage():
        pltpu.sync_copy(h2_hbm, tab_sh)

    plsc.subcore_barrier()

    def fire(b):
        hs = []
        for j in range(FPC):
            hs.append(pltpu.async_copy(
                tab_sh.at[idx_v.at[b, j]],
                rows_v.at[b, pl.ds(j * FIRE, FIRE)],
                g_sem))
        return hs

    pltpu.sync_copy(inds_hbm.at[pl.ds(irow0, FPC)], idx_v.at[0])
    gh = fire(0)
    ih = pltpu.async_copy(inds_hbm.at[pl.ds(irow0 + FPC, FPC)],
                          idx_v.at[1], idx_sem)

    zeros = jnp.zeros((16,), jnp.float32)
    s0, s1, q0, q1 = zeros, zeros, zeros, zeros
    for g in range(NCH):
        b = g % 2
        for h in gh:
            h.wait()
        if g + 1 < NCH:
            ih.wait()
            gh = fire((g + 1) % 2)
            if g + 2 < NCH:
                ih = pltpu.async_copy(
                    inds_hbm.at[pl.ds(irow0 + (g + 2) * FPC, FPC)],
                    idx_v.at[b], idx_sem)

        def node(i, carry):
            s0, s1, q0, q1 = carry
            base = i * K
            r0, r1 = plsc.unpack(rows_v[b, base],
                                 format=plsc.PackFormat.INTERLEAVED)
            mx0, mn0, mx1, mn1 = r0, r0, r1, r1
            s0 = s0 + r0
            s1 = s1 + r1
            q0 = q0 + r0 * r0
            q1 = q1 + r1 * r1
            for j in range(1, K):
                r0, r1 = plsc.unpack(rows_v[b, base + j],
                                     format=plsc.PackFormat.INTERLEAVED)
                mx0 = jnp.maximum(mx0, r0)
                mn0 = jnp.minimum(mn0, r0)
                mx1 = jnp.maximum(mx1, r1)
                mn1 = jnp.minimum(mn1, r1)
                s0 = s0 + r0
                s1 = s1 + r1
                q0 = q0 + r0 * r0
                q1 = q1 + r1 * r1
            obx_v[b, i, pl.ds(0, 16)] = mx0
            obx_v[b, i, pl.ds(16, 16)] = mx1
            obn_v[b, i, pl.ds(0, 16)] = mn0
            obn_v[b, i, pl.ds(16, 16)] = mn1
            return (s0, s1, q0, q1)

        s0, s1, q0, q1 = lax.fori_loop(0, C, node, (s0, s1, q0, q1))
        pltpu.sync_copy(obx_v.at[b], mx_hbm.at[pl.ds(node0 + g * C, C)])
        pltpu.sync_copy(obn_v.at[b], mn_hbm.at[pl.ds(node0 + g * C, C)])

    acc_v[0, pl.ds(0, 16)] = s0
    acc_v[0, pl.ds(16, 16)] = s1
    acc_v[0, pl.ds(32, 16)] = q0
    acc_v[0, pl.ds(48, 16)] = q1
    zz = jnp.zeros((16,), jnp.float32)
    for r in range(1, 8):
        for c in range(4):
            acc_v[r, pl.ds(c * 16, 16)] = zz
    pltpu.sync_copy(acc_v, st_hbm.at[pl.ds(w * 8, 8)])


_sc_call = functools.partial(
    pl.kernel,
    mesh=plsc.VectorSubcoreMesh(core_axis_name="c", subcore_axis_name="s"),
    out_type=[
        jax.ShapeDtypeStruct((NPAD, HID), jnp.float32),
        jax.ShapeDtypeStruct((NPAD, HID), jnp.float32),
        jax.ShapeDtypeStruct((NW * 8, 2 * HID), jnp.float32),
    ],
    scratch_types=[
        pltpu.VMEM((2, FPC, FIRE), jnp.int32),
        pltpu.VMEM((2, C * K, HID), jnp.bfloat16),
        pltpu.VMEM((2, C, HID), jnp.float32),
        pltpu.VMEM((2, C, HID), jnp.float32),
        pltpu.VMEM((8, 2 * HID), jnp.float32),
        pltpu.VMEM_SHARED((NT, HID), jnp.bfloat16),
        pltpu.SemaphoreType.DMA,
        pltpu.SemaphoreType.DMA,
    ],
    compiler_params=pltpu.CompilerParams(use_tc_tiling_on_sc=False,
                                         needs_layout_passes=False),
)(_sc_body)


def kernel(q_points, s_points, neighb_inds, x,
           W1, bn1_g, bn1_b, Wc, bnc_g, bnc_b, bno_g, bno_b, W2, bn2_g, bn2_b):
    h2 = pl.pallas_call(
        _tc1_body,
        out_shape=jax.ShapeDtypeStruct((NT, HID), jnp.bfloat16),
    )(x, W1, bn1_g.reshape(1, HID), bn1_b.reshape(1, HID), Wc)

    inds = jnp.pad(neighb_inds, ((0, NPAD - N), (0, 0)), constant_values=N)
    inds = inds.reshape(NPAD * K // FIRE, FIRE)
    mx, mn, st = _sc_call(h2, inds)

    out = pl.pallas_call(
        _tc2_body,
        out_shape=jax.ShapeDtypeStruct((N, OUT_DIM), jnp.float32),
    )(mx[:N], mn[:N], st, x,
      bnc_g[PERM].reshape(1, HID), bnc_b[PERM].reshape(1, HID),
      bno_g[PERM].reshape(1, HID), bno_b[PERM].reshape(1, HID),
      W2[:, PERM], bn2_g.reshape(1, OUT_DIM), bn2_b.reshape(1, OUT_DIM))
    return out


# speedup vs baseline: 8.6380x; 1.0936x over previous
import functools

import jax
import jax.numpy as jnp
import numpy as np
from jax import lax
from jax.experimental import pallas as pl
from jax.experimental.pallas import tpu as pltpu
from jax.experimental.pallas import tpu_sc as plsc

N = 10000
K = 32
IN_DIM = 128
OUT_DIM = 128
HID = 32
EPS = 1e-5

NW = 32
NPAD = 10240
PW = NPAD // NW
C = 32
NCH = PW // C
FIRE = 128
FPC = C * K // FIRE
NT = N + 16

ILV = np.empty((HID,), np.int64)
ILV[0::2] = np.arange(0, HID // 2)
ILV[1::2] = np.arange(HID // 2, HID)


def _lrelu(v):
    return jnp.where(v >= 0, v, 0.1 * v)


def _tc1_body(x_ref, w1_ref, g1_ref, b1_ref, wce_ref, out_ref):
    x = x_ref[...]
    h = lax.dot_general(x, w1_ref[...], (((1,), (1,)), ((), ())),
                        preferred_element_type=jnp.float32)
    m = jnp.mean(h, axis=0, keepdims=True)
    v = jnp.mean((h - m) * (h - m), axis=0, keepdims=True)
    h = _lrelu((h - m) / jnp.sqrt(v + EPS) * g1_ref[...] + b1_ref[...])
    h2 = lax.dot_general(h, wce_ref[...], (((1,), (1,)), ((), ())),
                         preferred_element_type=jnp.float32)
    out_ref[:N, :] = h2.astype(jnp.bfloat16)
    out_ref[N:, :] = jnp.zeros((NT - N, HID), jnp.bfloat16)


def _tc2_body(mxs_ref, st_ref, x_ref, sg_ref, cg_ref, cb_ref, og_ref, ob_ref,
              w2_ref, g2_ref, b2_ref, out_ref):
    sg = sg_ref[...]
    s = jnp.sum(st_ref[...], axis=0, keepdims=True)
    cnt = float(N * K)
    meanc = sg * (s[:, :HID] / cnt)
    varc = s[:, HID:] / cnt - meanc * meanc
    rstd = 1.0 / jnp.sqrt(varc + EPS)
    scale = cg_ref[...] * rstd
    shift = cb_ref[...] - meanc * scale
    v = _lrelu(mxs_ref[:N] * (sg * scale) + shift)
    m = jnp.mean(v, axis=0, keepdims=True)
    var = jnp.mean((v - m) * (v - m), axis=0, keepdims=True)
    v = _lrelu((v - m) / jnp.sqrt(var + EPS) * og_ref[...] + ob_ref[...])
    y = lax.dot_general(v, w2_ref[...], (((1,), (1,)), ((), ())),
                        preferred_element_type=jnp.float32)
    m2 = jnp.mean(y, axis=0, keepdims=True)
    var2 = jnp.mean((y - m2) * (y - m2), axis=0, keepdims=True)
    y = (y - m2) / jnp.sqrt(var2 + EPS) * g2_ref[...] + b2_ref[...]
    out_ref[...] = _lrelu(y + x_ref[...])


def _sc_body(h2_hbm, inds_hbm, mx_hbm, st_hbm,
             idx_v, rows_v, obx_v, acc_v, tab_sh, idx_sem, g_sem):
    cid = lax.axis_index("c")
    sid = lax.axis_index("s")
    w = sid * 2 + cid
    node0 = w * PW
    irow0 = w * (PW * K // FIRE)

    @pl.when(sid == 0)
    def _stage():
        pltpu.sync_copy(h2_hbm, tab_sh)

    plsc.subcore_barrier()

    def fire(b):
        hs = []
        for j in range(FPC):
            hs.append(pltpu.async_copy(
                tab_sh.at[idx_v.at[b, j]],
                rows_v.at[b, pl.ds(j * FIRE, FIRE)],
                g_sem))
        return hs

    pltpu.sync_copy(inds_hbm.at[pl.ds(irow0, FPC)], idx_v.at[0])
    gh = fire(0)
    ih = pltpu.async_copy(inds_hbm.at[pl.ds(irow0 + FPC, FPC)],
                          idx_v.at[1], idx_sem)

    zeros = jnp.zeros((16,), jnp.float32)
    s0, s1, q0, q1 = zeros, zeros, zeros, zeros
    for g in range(NCH):
        b = g % 2
        for h in gh:
            h.wait()
        if g + 1 < NCH:
            ih.wait()
            gh = fire((g + 1) % 2)
            if g + 2 < NCH:
                ih = pltpu.async_copy(
                    inds_hbm.at[pl.ds(irow0 + (g + 2) * FPC, FPC)],
                    idx_v.at[b], idx_sem)

        def node(i, carry):
            s0, s1, q0, q1 = carry
            base = i * K
            r0, r1 = plsc.unpack(rows_v[b, base],
                                 format=plsc.PackFormat.INTERLEAVED)
            mx0, mx1 = r0, r1
            s0 = s0 + r0
            s1 = s1 + r1
            q0 = q0 + r0 * r0
            q1 = q1 + r1 * r1
            for j in range(1, K):
                r0, r1 = plsc.unpack(rows_v[b, base + j],
                                     format=plsc.PackFormat.INTERLEAVED)
                mx0 = jnp.maximum(mx0, r0)
                mx1 = jnp.maximum(mx1, r1)
                s0 = s0 + r0
                s1 = s1 + r1
                q0 = q0 + r0 * r0
                q1 = q1 + r1 * r1
            obx_v[b, i, pl.ds(0, 16)] = mx0
            obx_v[b, i, pl.ds(16, 16)] = mx1
            return (s0, s1, q0, q1)

        s0, s1, q0, q1 = lax.fori_loop(0, C, node, (s0, s1, q0, q1))
        pltpu.sync_copy(obx_v.at[b], mx_hbm.at[pl.ds(node0 + g * C, C)])

    acc_v[0, pl.ds(0, 16)] = s0
    acc_v[0, pl.ds(16, 16)] = s1
    acc_v[0, pl.ds(32, 16)] = q0
    acc_v[0, pl.ds(48, 16)] = q1
    zz = jnp.zeros((16,), jnp.float32)
    for r in range(1, 8):
        for c in range(4):
            acc_v[r, pl.ds(c * 16, 16)] = zz
    pltpu.sync_copy(acc_v, st_hbm.at[pl.ds(w * 8, 8)])


_sc_call = functools.partial(
    pl.kernel,
    mesh=plsc.VectorSubcoreMesh(core_axis_name="c", subcore_axis_name="s"),
    out_type=[
        jax.ShapeDtypeStruct((NPAD, HID), jnp.float32),
        jax.ShapeDtypeStruct((NW * 8, 2 * HID), jnp.float32),
    ],
    scratch_types=[
        pltpu.VMEM((2, FPC, FIRE), jnp.int32),
        pltpu.VMEM((2, C * K, HID), jnp.bfloat16),
        pltpu.VMEM((2, C, HID), jnp.float32),
        pltpu.VMEM((8, 2 * HID), jnp.float32),
        pltpu.VMEM_SHARED((NT, HID), jnp.bfloat16),
        pltpu.SemaphoreType.DMA,
        pltpu.SemaphoreType.DMA,
    ],
    compiler_params=pltpu.CompilerParams(use_tc_tiling_on_sc=False,
                                         needs_layout_passes=False),
)(_sc_body)


def kernel(q_points, s_points, neighb_inds, x,
           W1, bn1_g, bn1_b, Wc, bnc_g, bnc_b, bno_g, bno_b, W2, bn2_g, bn2_b):
    sg = jnp.sign(bnc_g)
    wc_eff = (Wc * sg[:, None])[ILV]
    h2 = pl.pallas_call(
        _tc1_body,
        out_shape=jax.ShapeDtypeStruct((NT, HID), jnp.bfloat16),
    )(x, W1, bn1_g.reshape(1, HID), bn1_b.reshape(1, HID), wc_eff)

    inds = jnp.pad(neighb_inds, ((0, NPAD - N), (0, 0)), constant_values=N)
    inds = inds.reshape(NPAD * K // FIRE, FIRE)
    mxs, st = _sc_call(h2, inds)

    out = pl.pallas_call(
        _tc2_body,
        out_shape=jax.ShapeDtypeStruct((N, OUT_DIM), jnp.float32),
    )(mxs, st, x, sg.reshape(1, HID),
      bnc_g.reshape(1, HID), bnc_b.reshape(1, HID),
      bno_g.reshape(1, HID), bno_b.reshape(1, HID),
      W2, bn2_g.reshape(1, OUT_DIM), bn2_b.reshape(1, OUT_DIM))
    return out


# speedup vs baseline: 9.7795x; 1.1321x over previous
import functools

import jax
import jax.numpy as jnp
import numpy as np
from jax import lax
from jax.experimental import pallas as pl
from jax.experimental.pallas import tpu as pltpu
from jax.experimental.pallas import tpu_sc as plsc

N = 10000
K = 32
IN_DIM = 128
OUT_DIM = 128
HID = 32
EPS = 1e-5

NW = 32
NPAD = 10240
PW = NPAD // NW
C = 32
NCH = PW // C
FIRE = 128
FPC = C * K // FIRE
NT = N + 16

ILV = np.empty((HID,), np.int64)
ILV[0::2] = np.arange(0, HID // 2)
ILV[1::2] = np.arange(HID // 2, HID)


def _lrelu(v):
    return jnp.where(v >= 0, v, 0.1 * v)


def _tc1_body(x_ref, w1_ref, g1_ref, b1_ref, wce_ref, out_ref):
    x = x_ref[...]
    h = lax.dot_general(x, w1_ref[...], (((1,), (1,)), ((), ())),
                        preferred_element_type=jnp.float32)
    m = jnp.mean(h, axis=0, keepdims=True)
    v = jnp.mean((h - m) * (h - m), axis=0, keepdims=True)
    h = _lrelu((h - m) / jnp.sqrt(v + EPS) * g1_ref[...] + b1_ref[...])
    h2 = lax.dot_general(h, wce_ref[...], (((1,), (1,)), ((), ())),
                         preferred_element_type=jnp.float32)
    out_ref[:N, :] = h2.astype(jnp.bfloat16)
    out_ref[N:, :] = jnp.zeros((NT - N, HID), jnp.bfloat16)


def _tc2_body(mxs_ref, st_ref, x_ref, sg_ref, cg_ref, cb_ref, og_ref, ob_ref,
              w2_ref, g2_ref, b2_ref, out_ref):
    sg = sg_ref[...]
    s = jnp.sum(st_ref[...], axis=0, keepdims=True)
    cnt = float(N * K)
    meanc = sg * (s[:, :HID] / cnt)
    varc = s[:, HID:] / cnt - meanc * meanc
    rstd = 1.0 / jnp.sqrt(varc + EPS)
    scale = cg_ref[...] * rstd
    shift = cb_ref[...] - meanc * scale
    v = _lrelu(mxs_ref[:N] * (sg * scale) + shift)
    m = jnp.mean(v, axis=0, keepdims=True)
    var = jnp.mean((v - m) * (v - m), axis=0, keepdims=True)
    v = _lrelu((v - m) / jnp.sqrt(var + EPS) * og_ref[...] + ob_ref[...])
    y = lax.dot_general(v, w2_ref[...], (((1,), (1,)), ((), ())),
                        preferred_element_type=jnp.float32)
    m2 = jnp.mean(y, axis=0, keepdims=True)
    var2 = jnp.mean((y - m2) * (y - m2), axis=0, keepdims=True)
    y = (y - m2) / jnp.sqrt(var2 + EPS) * g2_ref[...] + b2_ref[...]
    out_ref[...] = _lrelu(y + x_ref[...])


def _sc_body(h2_hbm, inds_hbm, mx_hbm, st_hbm,
             idx_v, rows_v, obx_v, acc_v, tab_sh, idx_sem, g_sem):
    cid = lax.axis_index("c")
    sid = lax.axis_index("s")
    w = sid * 2 + cid
    node0 = w * PW
    irow0 = w * (PW * K // FIRE)

    @pl.when(sid == 0)
    def _stage():
        pltpu.sync_copy(h2_hbm, tab_sh)

    plsc.subcore_barrier()

    def fire(b):
        hs = []
        for j in range(FPC):
            hs.append(pltpu.async_copy(
                tab_sh.at[idx_v.at[b, j]],
                rows_v.at[b, pl.ds(j * FIRE, FIRE)],
                g_sem))
        return hs

    pltpu.sync_copy(inds_hbm.at[pl.ds(irow0, FPC)], idx_v.at[0])
    gh = fire(0)
    ih = pltpu.async_copy(inds_hbm.at[pl.ds(irow0 + FPC, FPC)],
                          idx_v.at[1], idx_sem)

    zeros = jnp.zeros((16,), jnp.float32)
    s0, s1, q0, q1 = zeros, zeros, zeros, zeros
    for g in range(NCH):
        b = g % 2
        for h in gh:
            h.wait()
        if g + 1 < NCH:
            ih.wait()
            gh = fire((g + 1) % 2)
            if g + 2 < NCH:
                ih = pltpu.async_copy(
                    inds_hbm.at[pl.ds(irow0 + (g + 2) * FPC, FPC)],
                    idx_v.at[b], idx_sem)

        def node(i, carry):
            s0, s1, q0, q1 = carry
            base = i * K
            r = rows_v[b, base]
            mx = r
            sm = r
            sq = r * r
            for j in range(1, K):
                r = rows_v[b, base + j]
                mx = jnp.maximum(mx, r)
                sm = sm + r
                sq = sq + r * r
            mx0, mx1 = plsc.unpack(mx, format=plsc.PackFormat.INTERLEAVED)
            a0, a1 = plsc.unpack(sm, format=plsc.PackFormat.INTERLEAVED)
            c0, c1 = plsc.unpack(sq, format=plsc.PackFormat.INTERLEAVED)
            s0 = s0 + a0
            s1 = s1 + a1
            q0 = q0 + c0
            q1 = q1 + c1
            obx_v[b, i, pl.ds(0, 16)] = mx0
            obx_v[b, i, pl.ds(16, 16)] = mx1
            return (s0, s1, q0, q1)

        s0, s1, q0, q1 = lax.fori_loop(0, C, node, (s0, s1, q0, q1))
        pltpu.sync_copy(obx_v.at[b], mx_hbm.at[pl.ds(node0 + g * C, C)])

    acc_v[0, pl.ds(0, 16)] = s0
    acc_v[0, pl.ds(16, 16)] = s1
    acc_v[0, pl.ds(32, 16)] = q0
    acc_v[0, pl.ds(48, 16)] = q1
    zz = jnp.zeros((16,), jnp.float32)
    for r in range(1, 8):
        for c in range(4):
            acc_v[r, pl.ds(c * 16, 16)] = zz
    pltpu.sync_copy(acc_v, st_hbm.at[pl.ds(w * 8, 8)])


_sc_call = functools.partial(
    pl.kernel,
    mesh=plsc.VectorSubcoreMesh(core_axis_name="c", subcore_axis_name="s"),
    out_type=[
        jax.ShapeDtypeStruct((NPAD, HID), jnp.float32),
        jax.ShapeDtypeStruct((NW * 8, 2 * HID), jnp.float32),
    ],
    scratch_types=[
        pltpu.VMEM((2, FPC, FIRE), jnp.int32),
        pltpu.VMEM((2, C * K, HID), jnp.bfloat16),
        pltpu.VMEM((2, C, HID), jnp.float32),
        pltpu.VMEM((8, 2 * HID), jnp.float32),
        pltpu.VMEM_SHARED((NT, HID), jnp.bfloat16),
        pltpu.SemaphoreType.DMA,
        pltpu.SemaphoreType.DMA,
    ],
    compiler_params=pltpu.CompilerParams(use_tc_tiling_on_sc=False,
                                         needs_layout_passes=False),
)(_sc_body)


def kernel(q_points, s_points, neighb_inds, x,
           W1, bn1_g, bn1_b, Wc, bnc_g, bnc_b, bno_g, bno_b, W2, bn2_g, bn2_b):
    sg = jnp.sign(bnc_g)
    wc_eff = (Wc * sg[:, None])[ILV]
    h2 = pl.pallas_call(
        _tc1_body,
        out_shape=jax.ShapeDtypeStruct((NT, HID), jnp.bfloat16),
    )(x, W1, bn1_g.reshape(1, HID), bn1_b.reshape(1, HID), wc_eff)

    inds = jnp.pad(neighb_inds, ((0, NPAD - N), (0, 0)), constant_values=N)
    inds = inds.reshape(NPAD * K // FIRE, FIRE)
    mxs, st = _sc_call(h2, inds)

    out = pl.pallas_call(
        _tc2_body,
        out_shape=jax.ShapeDtypeStruct((N, OUT_DIM), jnp.float32),
    )(mxs, st, x, sg.reshape(1, HID),
      bnc_g.reshape(1, HID), bnc_b.reshape(1, HID),
      bno_g.reshape(1, HID), bno_b.reshape(1, HID),
      W2, bn2_g.reshape(1, OUT_DIM), bn2_b.reshape(1, OUT_DIM))
    return out


# speedup vs baseline: 10.1934x; 1.0423x over previous
import functools

import jax
import jax.numpy as jnp
import numpy as np
from jax import lax
from jax.experimental import pallas as pl
from jax.experimental.pallas import tpu as pltpu
from jax.experimental.pallas import tpu_sc as plsc

N = 10000
K = 32
IN_DIM = 128
OUT_DIM = 128
HID = 32
EPS = 1e-5

NW = 32
NPAD = 10240
PW = NPAD // NW
C = 32
NCH = PW // C
FIRE = 128
FPC = C * K // FIRE
NT = N + 16
NF = N // 4

_ILV = np.empty((HID,), np.int64)
_ILV[0::2] = np.arange(0, HID // 2)
_ILV[1::2] = np.arange(HID // 2, HID)
_PERM_M = np.zeros((HID, HID), np.float32)
_PERM_M[np.arange(HID), _ILV] = 1.0


def _lrelu(v):
    return jnp.where(v >= 0, v, 0.1 * v)


def _tc1_body(x_ref, w1_ref, g1_ref, b1_ref, wc_ref, cg_ref, pm_ref, out_ref):
    x = x_ref[...]
    h = lax.dot_general(x, w1_ref[...], (((1,), (1,)), ((), ())),
                        preferred_element_type=jnp.float32)
    m = jnp.mean(h, axis=0, keepdims=True)
    v = jnp.mean((h - m) * (h - m), axis=0, keepdims=True)
    h = _lrelu((h - m) / jnp.sqrt(v + EPS) * g1_ref[...] + b1_ref[...])
    wc_s = wc_ref[...] * jnp.sign(cg_ref[...]).reshape(HID, 1)
    wc_eff = jnp.dot(pm_ref[...], wc_s, preferred_element_type=jnp.float32)
    h2 = lax.dot_general(h, wc_eff, (((1,), (1,)), ((), ())),
                         preferred_element_type=jnp.float32)
    out_ref[:N, :] = h2.astype(jnp.bfloat16)
    out_ref[N:, :] = jnp.zeros((NT - N, HID), jnp.bfloat16)


def _tc2_body(mxf_ref, st_ref, x_ref, cg_ref, cb_ref, og_ref, ob_ref,
              w2_ref, g2_ref, b2_ref, out_ref):
    def x4(a):
        return jnp.concatenate([a, a, a, a], axis=1)

    def fold4(a):
        return (a[:, 0 * HID:1 * HID] + a[:, 1 * HID:2 * HID]
                + a[:, 2 * HID:3 * HID] + a[:, 3 * HID:4 * HID])

    cg = cg_ref[...]
    sg = jnp.sign(cg)
    s = jnp.sum(st_ref[...], axis=0, keepdims=True)
    cnt = float(N * K)
    meanc = sg * (s[:, :HID] / cnt)
    varc = s[:, HID:] / cnt - meanc * meanc
    rstd = 1.0 / jnp.sqrt(varc + EPS)
    scale = cg * rstd
    shift = cb_ref[...] - meanc * scale
    mz = mxf_ref[:NF]
    v = _lrelu(mz * x4(sg * scale) + x4(shift))
    m = fold4(jnp.sum(v, axis=0, keepdims=True)) / N
    q = fold4(jnp.sum(v * v, axis=0, keepdims=True)) / N
    var = q - m * m
    v = _lrelu((v - x4(m)) / jnp.sqrt(x4(var) + EPS) * x4(og_ref[...])
               + x4(ob_ref[...]))
    w2 = w2_ref[...]
    ys = [lax.dot_general(v[:, j * HID:(j + 1) * HID], w2,
                          (((1,), (1,)), ((), ())),
                          preferred_element_type=jnp.float32)
          for j in range(4)]
    y = jnp.stack(ys, axis=1).reshape(N, OUT_DIM)
    m2 = jnp.mean(y, axis=0, keepdims=True)
    var2 = jnp.mean((y - m2) * (y - m2), axis=0, keepdims=True)
    y = (y - m2) / jnp.sqrt(var2 + EPS) * g2_ref[...] + b2_ref[...]
    out_ref[...] = _lrelu(y + x_ref[...])


def _sc_body(h2_hbm, inds_hbm, mx_hbm, st_hbm,
             idx_v, rows_v, obx_v, acc_v, tab_sh, idx_sem, g_sem):
    cid = lax.axis_index("c")
    sid = lax.axis_index("s")
    w = sid * 2 + cid
    orow0 = w * (PW // 4)
    irow0 = w * (PW * K // FIRE)

    @pl.when(sid == 0)
    def _stage():
        pltpu.sync_copy(h2_hbm, tab_sh)

    plsc.subcore_barrier()

    def fire(b):
        hs = []
        for j in range(FPC):
            hs.append(pltpu.async_copy(
                tab_sh.at[idx_v.at[b, j]],
                rows_v.at[b, pl.ds(j * FIRE, FIRE)],
                g_sem))
        return hs

    pltpu.sync_copy(inds_hbm.at[pl.ds(irow0, FPC)], idx_v.at[0])
    gh = fire(0)
    ih = pltpu.async_copy(inds_hbm.at[pl.ds(irow0 + FPC, FPC)],
                          idx_v.at[1], idx_sem)

    zeros = jnp.zeros((16,), jnp.float32)
    s0, s1, q0, q1 = zeros, zeros, zeros, zeros
    for g in range(NCH):
        b = g % 2
        for h in gh:
            h.wait()
        if g + 1 < NCH:
            ih.wait()
            gh = fire((g + 1) % 2)
            if g + 2 < NCH:
                ih = pltpu.async_copy(
                    inds_hbm.at[pl.ds(irow0 + (g + 2) * FPC, FPC)],
                    idx_v.at[b], idx_sem)

        def node(i, carry):
            s0, s1, q0, q1 = carry
            base = i * K
            r = rows_v[b, base]
            mx = r
            sm = r
            sq = r * r
            for j in range(1, K):
                r = rows_v[b, base + j]
                mx = jnp.maximum(mx, r)
                sm = sm + r
                sq = sq + r * r
            mx0, mx1 = plsc.unpack(mx, format=plsc.PackFormat.INTERLEAVED)
            a0, a1 = plsc.unpack(sm, format=plsc.PackFormat.INTERLEAVED)
            c0, c1 = plsc.unpack(sq, format=plsc.PackFormat.INTERLEAVED)
            s0 = s0 + a0
            s1 = s1 + a1
            q0 = q0 + c0
            q1 = q1 + c1
            col = (i & 3) * HID
            obx_v[b, i >> 2, pl.ds(col, 16)] = mx0
            obx_v[b, i >> 2, pl.ds(col + 16, 16)] = mx1
            return (s0, s1, q0, q1)

        s0, s1, q0, q1 = lax.fori_loop(0, C, node, (s0, s1, q0, q1))
        pltpu.sync_copy(obx_v.at[b], mx_hbm.at[pl.ds(orow0 + g * (C // 4),
                                                     C // 4)])

    acc_v[0, pl.ds(0, 16)] = s0
    acc_v[0, pl.ds(16, 16)] = s1
    acc_v[0, pl.ds(32, 16)] = q0
    acc_v[0, pl.ds(48, 16)] = q1
    zz = jnp.zeros((16,), jnp.float32)
    for r in range(1, 8):
        for c in range(4):
            acc_v[r, pl.ds(c * 16, 16)] = zz
    pltpu.sync_copy(acc_v, st_hbm.at[pl.ds(w * 8, 8)])


_sc_call = functools.partial(
    pl.kernel,
    mesh=plsc.VectorSubcoreMesh(core_axis_name="c", subcore_axis_name="s"),
    out_type=[
        jax.ShapeDtypeStruct((NPAD // 4, 4 * HID), jnp.float32),
        jax.ShapeDtypeStruct((NW * 8, 2 * HID), jnp.float32),
    ],
    scratch_types=[
        pltpu.VMEM((2, FPC, FIRE), jnp.int32),
        pltpu.VMEM((2, C * K, HID), jnp.bfloat16),
        pltpu.VMEM((2, C // 4, 4 * HID), jnp.float32),
        pltpu.VMEM((8, 2 * HID), jnp.float32),
        pltpu.VMEM_SHARED((NT, HID), jnp.bfloat16),
        pltpu.SemaphoreType.DMA,
        pltpu.SemaphoreType.DMA,
    ],
    compiler_params=pltpu.CompilerParams(use_tc_tiling_on_sc=False,
                                         needs_layout_passes=False),
)(_sc_body)


def kernel(q_points, s_points, neighb_inds, x,
           W1, bn1_g, bn1_b, Wc, bnc_g, bnc_b, bno_g, bno_b, W2, bn2_g, bn2_b):
    h2 = pl.pallas_call(
        _tc1_body,
        out_shape=jax.ShapeDtypeStruct((NT, HID), jnp.bfloat16),
    )(x, W1, bn1_g.reshape(1, HID), bn1_b.reshape(1, HID), Wc,
      bnc_g.reshape(1, HID), jnp.asarray(_PERM_M))

    inds = jnp.pad(neighb_inds, ((0, NPAD - N), (0, 0)), constant_values=N)
    inds = inds.reshape(NPAD * K // FIRE, FIRE)
    mxf, st = _sc_call(h2, inds)

    out = pl.pallas_call(
        _tc2_body,
        out_shape=jax.ShapeDtypeStruct((N, OUT_DIM), jnp.float32),
    )(mxf, st, x,
      bnc_g.reshape(1, HID), bnc_b.reshape(1, HID),
      bno_g.reshape(1, HID), bno_b.reshape(1, HID),
      W2, bn2_g.reshape(1, OUT_DIM), bn2_b.reshape(1, OUT_DIM))
    return out


# speedup vs baseline: 11.5336x; 1.1315x over previous
import functools

import jax
import jax.numpy as jnp
import numpy as np
from jax import lax
from jax.experimental import pallas as pl
from jax.experimental.pallas import tpu as pltpu
from jax.experimental.pallas import tpu_sc as plsc

N = 10000
K = 32
IN_DIM = 128
OUT_DIM = 128
HID = 32
EPS = 1e-5

NW = 32
NPAD = 10240
PW = NPAD // NW
C = 32
NCH = PW // C
FIRE = 128
FPC = C * K // FIRE
NT = N + 16
NF = N // 4

_ILV = np.empty((HID,), np.int64)
_ILV[0::2] = np.arange(0, HID // 2)
_ILV[1::2] = np.arange(HID // 2, HID)
_PERM_M = np.zeros((HID, HID), np.float32)
_PERM_M[np.arange(HID), _ILV] = 1.0


def _lrelu(v):
    return jnp.where(v >= 0, v, 0.1 * v)


def _tc1_body(x_ref, w1_ref, g1_ref, b1_ref, wc_ref, cg_ref, pm_ref, out_ref):
    x = x_ref[...]
    h = lax.dot_general(x, w1_ref[...], (((1,), (1,)), ((), ())),
                        preferred_element_type=jnp.float32)
    m = jnp.mean(h, axis=0, keepdims=True)
    v = jnp.mean((h - m) * (h - m), axis=0, keepdims=True)
    h = _lrelu((h - m) / jnp.sqrt(v + EPS) * g1_ref[...] + b1_ref[...])
    wc_s = wc_ref[...] * jnp.sign(cg_ref[...]).reshape(HID, 1)
    wc_eff = jnp.dot(pm_ref[...], wc_s, preferred_element_type=jnp.float32)
    h2 = lax.dot_general(h, wc_eff, (((1,), (1,)), ((), ())),
                         preferred_element_type=jnp.float32)
    out_ref[:N, :] = h2.astype(jnp.bfloat16)
    out_ref[N:, :] = jnp.zeros((NT - N, HID), jnp.bfloat16)


BLK = NPAD // 4
TAIL = N - 3 * BLK


def _tc2_body(mxf_ref, st_ref, x_ref, cg_ref, cb_ref, og_ref, ob_ref,
              w2_ref, g2_ref, b2_ref, out_ref):
    def x4(a):
        return jnp.concatenate([a, a, a, a], axis=1)

    def fold4(a):
        return (a[:, 0 * HID:1 * HID] + a[:, 1 * HID:2 * HID]
                + a[:, 2 * HID:3 * HID] + a[:, 3 * HID:4 * HID])

    cg = cg_ref[...]
    sg = jnp.sign(cg)
    s = jnp.sum(st_ref[...], axis=0, keepdims=True)
    cnt = float(N * K)
    meanc = sg * (s[:, :HID] / cnt)
    varc = s[:, HID:] / cnt - meanc * meanc
    rstd = 1.0 / jnp.sqrt(varc + EPS)
    scale = cg * rstd
    shift = cb_ref[...] - meanc * scale
    mz = mxf_ref[...]
    v = _lrelu(mz * x4(sg * scale) + x4(shift))
    sf = jnp.sum(v[:TAIL], axis=0, keepdims=True)
    stl = jnp.sum(v[TAIL:, :3 * HID], axis=0, keepdims=True)
    qf = jnp.sum(v[:TAIL] * v[:TAIL], axis=0, keepdims=True)
    qtl = jnp.sum(v[TAIL:, :3 * HID] * v[TAIL:, :3 * HID],
                  axis=0, keepdims=True)
    pad3 = jnp.zeros((1, HID), jnp.float32)
    m = (fold4(sf) + fold4(jnp.concatenate([stl, pad3], axis=1))) / N
    q = (fold4(qf) + fold4(jnp.concatenate([qtl, pad3], axis=1))) / N
    var = q - m * m
    v = _lrelu((v - x4(m)) / jnp.sqrt(x4(var) + EPS) * x4(og_ref[...])
               + x4(ob_ref[...]))
    vn = jnp.concatenate([v[:, 0:HID], v[:, HID:2 * HID],
                          v[:, 2 * HID:3 * HID], v[:TAIL, 3 * HID:]], axis=0)
    w2 = w2_ref[...]
    mv = jnp.sum(vn, axis=0, keepdims=True) / N
    cov = lax.dot_general(vn, vn, (((0,), (0,)), ((), ())),
                          preferred_element_type=jnp.float32) / N
    m2 = lax.dot_general(mv, w2, (((1,), (1,)), ((), ())),
                         preferred_element_type=jnp.float32)
    t = lax.dot_general(w2, cov, (((1,), (1,)), ((), ())),
                        preferred_element_type=jnp.float32)
    var2 = jnp.sum(t * w2, axis=1, keepdims=True).reshape(1, OUT_DIM)
    var2 = var2 - m2 * m2
    s2 = g2_ref[...] / jnp.sqrt(var2 + EPS)
    beff = b2_ref[...] - m2 * s2
    w2s = w2 * s2.reshape(OUT_DIM, 1)
    y = lax.dot_general(vn, w2s, (((1,), (1,)), ((), ())),
                        preferred_element_type=jnp.float32)
    out_ref[...] = _lrelu(y + beff + x_ref[...])


def _sc_body(h2_hbm, inds_hbm, mx_hbm, st_hbm,
             idx_v, rows_v, obx_v, acc_v, tab_sh, idx_sem, g_sem):
    cid = lax.axis_index("c")
    sid = lax.axis_index("s")
    w = sid * 2 + cid
    ocol = (w // 8) * HID
    orow0 = (w % 8) * PW
    irow0 = w * (PW * K // FIRE)

    @pl.when(sid == 0)
    def _stage():
        pltpu.sync_copy(h2_hbm, tab_sh)

    plsc.subcore_barrier()

    def fire(b):
        hs = []
        for j in range(FPC):
            hs.append(pltpu.async_copy(
                tab_sh.at[idx_v.at[b, j]],
                rows_v.at[b, pl.ds(j * FIRE, FIRE)],
                g_sem))
        return hs

    pltpu.sync_copy(inds_hbm.at[pl.ds(irow0, FPC)], idx_v.at[0])
    gh = fire(0)
    ih = pltpu.async_copy(inds_hbm.at[pl.ds(irow0 + FPC, FPC)],
                          idx_v.at[1], idx_sem)

    zeros = jnp.zeros((16,), jnp.float32)
    s0, s1, q0, q1 = zeros, zeros, zeros, zeros
    for g in range(NCH):
        b = g % 2
        for h in gh:
            h.wait()
        if g + 1 < NCH:
            ih.wait()
            gh = fire((g + 1) % 2)
            if g + 2 < NCH:
                ih = pltpu.async_copy(
                    inds_hbm.at[pl.ds(irow0 + (g + 2) * FPC, FPC)],
                    idx_v.at[b], idx_sem)

        def node(i, carry):
            s0, s1, q0, q1 = carry
            base = i * K
            r = rows_v[b, base]
            mx = r
            sm = r
            sq = r * r
            for j in range(1, K):
                r = rows_v[b, base + j]
                mx = jnp.maximum(mx, r)
                sm = sm + r
                sq = sq + r * r
            mx0, mx1 = plsc.unpack(mx, format=plsc.PackFormat.INTERLEAVED)
            a0, a1 = plsc.unpack(sm, format=plsc.PackFormat.INTERLEAVED)
            c0, c1 = plsc.unpack(sq, format=plsc.PackFormat.INTERLEAVED)
            s0 = s0 + a0
            s1 = s1 + a1
            q0 = q0 + c0
            q1 = q1 + c1
            obx_v[b, i, pl.ds(0, 16)] = mx0
            obx_v[b, i, pl.ds(16, 16)] = mx1
            return (s0, s1, q0, q1)

        s0, s1, q0, q1 = lax.fori_loop(0, C, node, (s0, s1, q0, q1))
        pltpu.sync_copy(obx_v.at[b],
                        mx_hbm.at[pl.ds(orow0 + g * C, C), pl.ds(ocol, HID)])

    acc_v[0, pl.ds(0, 16)] = s0
    acc_v[0, pl.ds(16, 16)] = s1
    acc_v[0, pl.ds(32, 16)] = q0
    acc_v[0, pl.ds(48, 16)] = q1
    zz = jnp.zeros((16,), jnp.float32)
    for r in range(1, 8):
        for c in range(4):
            acc_v[r, pl.ds(c * 16, 16)] = zz
    pltpu.sync_copy(acc_v, st_hbm.at[pl.ds(w * 8, 8)])


_sc_call = functools.partial(
    pl.kernel,
    mesh=plsc.VectorSubcoreMesh(core_axis_name="c", subcore_axis_name="s"),
    out_type=[
        jax.ShapeDtypeStruct((NPAD // 4, 4 * HID), jnp.float32),
        jax.ShapeDtypeStruct((NW * 8, 2 * HID), jnp.float32),
    ],
    scratch_types=[
        pltpu.VMEM((2, FPC, FIRE), jnp.int32),
        pltpu.VMEM((2, C * K, HID), jnp.bfloat16),
        pltpu.VMEM((2, C, HID), jnp.float32),
        pltpu.VMEM((8, 2 * HID), jnp.float32),
        pltpu.VMEM_SHARED((NT, HID), jnp.bfloat16),
        pltpu.SemaphoreType.DMA,
        pltpu.SemaphoreType.DMA,
    ],
    compiler_params=pltpu.CompilerParams(use_tc_tiling_on_sc=False,
                                         needs_layout_passes=False),
)(_sc_body)


def kernel(q_points, s_points, neighb_inds, x,
           W1, bn1_g, bn1_b, Wc, bnc_g, bnc_b, bno_g, bno_b, W2, bn2_g, bn2_b):
    h2 = pl.pallas_call(
        _tc1_body,
        out_shape=jax.ShapeDtypeStruct((NT, HID), jnp.bfloat16),
    )(x, W1, bn1_g.reshape(1, HID), bn1_b.reshape(1, HID), Wc,
      bnc_g.reshape(1, HID), jnp.asarray(_PERM_M))

    inds = jnp.pad(neighb_inds, ((0, NPAD - N), (0, 0)), constant_values=N)
    inds = inds.reshape(NPAD * K // FIRE, FIRE)
    mxf, st = _sc_call(h2, inds)

    out = pl.pallas_call(
        _tc2_body,
        out_shape=jax.ShapeDtypeStruct((N, OUT_DIM), jnp.float32),
    )(mxf, st, x,
      bnc_g.reshape(1, HID), bnc_b.reshape(1, HID),
      bno_g.reshape(1, HID), bno_b.reshape(1, HID),
      W2, bn2_g.reshape(1, OUT_DIM), bn2_b.reshape(1, OUT_DIM))
    return out
